# baseline (speedup 1.0000x reference)
import functools

import jax
import jax.numpy as jnp
from jax import lax
from jax.experimental import pallas as pl
from jax.experimental.pallas import tpu as pltpu
from jax.experimental.pallas import tpu_sc as plsc

B, N, VOCAB, EMB = 16, 1024, 4096, 192


_NT = 256


def _argmax_body(x_ref, tok_ref):
    xb = x_ref[0]
    m = jnp.max(xb, axis=-1, keepdims=True)
    iota = lax.broadcasted_iota(jnp.int32, xb.shape, 1)
    idx = jnp.min(jnp.where(xb == m, iota, VOCAB), axis=-1)
    tok_ref[0, 0] = idx.astype(jnp.int32)


def _argmax_tokens(x):
    return pl.pallas_call(
        _argmax_body,
        grid=(B, N // _NT),
        in_specs=[pl.BlockSpec((1, _NT, VOCAB), lambda b, n: (b, n, 0))],
        out_specs=pl.BlockSpec((1, 1, _NT), lambda b, n: (b, 0, n)),
        out_shape=jax.ShapeDtypeStruct((B, 1, N), jnp.int32),
    )(x)


_NC, _NS, _L = 2, 16, 16
_NW = _NC * _NS
_ROWS = B * N
_BPW = _ROWS // _NW
_CHUNK = 128
_NCHUNK = _BPW // _CHUNK


def _make_sc_gather():
    mesh = plsc.VectorSubcoreMesh(core_axis_name="c", subcore_axis_name="s")

    @functools.partial(
        pl.kernel,
        mesh=mesh,
        out_type=jax.ShapeDtypeStruct((B, N, EMB), jnp.float32),
        scratch_types=[
            pltpu.VMEM((_NCHUNK, _CHUNK), jnp.int32),
            pltpu.VMEM((_BPW, EMB), jnp.float32),
            pltpu.SemaphoreType.DMA,
        ],
        compiler_params=pltpu.CompilerParams(use_tc_tiling_on_sc=False),
    )
    def sc_gather(table_hbm, idx_hbm, out_hbm, idx_v, rows_v, sem):
        wid = lax.axis_index("s") * _NC + lax.axis_index("c")
        b = wid // 2
        noff = (wid % 2) * _BPW
        for j in range(_NCHUNK):
            pltpu.sync_copy(
                idx_hbm.at[b, 0, pl.ds(noff + j * _CHUNK, _CHUNK)],
                idx_v.at[j],
            )
        copies = []
        for j in range(_NCHUNK):
            copies.append(
                pltpu.async_copy(
                    table_hbm.at[idx_v.at[j]],
                    rows_v.at[pl.ds(j * _CHUNK, _CHUNK)],
                    sem,
                )
            )
        for c in copies:
            c.wait()
        pltpu.sync_copy(rows_v, out_hbm.at[b, pl.ds(noff, _BPW)])

    return sc_gather


_SC_GATHER_CACHE = []


def kernel(x, dictionary):
    tokens = _argmax_tokens(x)
    if not _SC_GATHER_CACHE:
        _SC_GATHER_CACHE.append(_make_sc_gather())
    return _SC_GATHER_CACHE[0](dictionary, tokens)


# speedup vs baseline: 1.1253x; 1.1253x over previous
import functools

import jax
import jax.numpy as jnp
from jax import lax
from jax.experimental import pallas as pl
from jax.experimental.pallas import tpu as pltpu
from jax.experimental.pallas import tpu_sc as plsc

B, N, VOCAB, EMB = 16, 1024, 4096, 192


_NT = 512


def _argmax_body(x_ref, tok_ref):
    xb = x_ref[0]
    m = jnp.max(xb, axis=-1, keepdims=True)
    iota = lax.broadcasted_iota(jnp.int32, xb.shape, 1)
    idx = jnp.min(jnp.where(xb == m, iota, VOCAB), axis=-1)
    tok_ref[0, 0] = idx.astype(jnp.int32)


def _argmax_tokens(x):
    return pl.pallas_call(
        _argmax_body,
        grid=(B, N // _NT),
        in_specs=[pl.BlockSpec((1, _NT, VOCAB), lambda b, n: (b, n, 0))],
        out_specs=pl.BlockSpec((1, 1, _NT), lambda b, n: (b, 0, n)),
        out_shape=jax.ShapeDtypeStruct((B, 1, N), jnp.int32),
    )(x)


_NC, _NS, _L = 2, 16, 16
_NW = _NC * _NS
_ROWS = B * N
_BPW = _ROWS // _NW
_CHUNK = 128
_NCHUNK = _BPW // _CHUNK


def _make_sc_gather():
    mesh = plsc.VectorSubcoreMesh(core_axis_name="c", subcore_axis_name="s")

    @functools.partial(
        pl.kernel,
        mesh=mesh,
        out_type=jax.ShapeDtypeStruct((B, N, EMB), jnp.float32),
        scratch_types=[
            pltpu.VMEM((_NCHUNK, _CHUNK), jnp.int32),
            pltpu.VMEM((_BPW, EMB), jnp.float32),
            pltpu.SemaphoreType.DMA,
        ],
        compiler_params=pltpu.CompilerParams(use_tc_tiling_on_sc=False),
    )
    def sc_gather(table_hbm, idx_hbm, out_hbm, idx_v, rows_v, sem):
        wid = lax.axis_index("s") * _NC + lax.axis_index("c")
        b = wid // 2
        noff = (wid % 2) * _BPW
        for j in range(_NCHUNK):
            pltpu.sync_copy(
                idx_hbm.at[b, 0, pl.ds(noff + j * _CHUNK, _CHUNK)],
                idx_v.at[j],
            )
        copies = []
        for j in range(_NCHUNK):
            copies.append(
                pltpu.async_copy(
                    table_hbm.at[idx_v.at[j]],
                    rows_v.at[pl.ds(j * _CHUNK, _CHUNK)],
                    sem,
                )
            )
        for c in copies:
            c.wait()
        pltpu.sync_copy(rows_v, out_hbm.at[b, pl.ds(noff, _BPW)])

    return sc_gather


_SC_GATHER_CACHE = []


def kernel(x, dictionary):
    tokens = _argmax_tokens(x)
    if not _SC_GATHER_CACHE:
        _SC_GATHER_CACHE.append(_make_sc_gather())
    return _SC_GATHER_CACHE[0](dictionary, tokens)


# speedup vs baseline: 1.1752x; 1.0443x over previous
import functools

import jax
import jax.numpy as jnp
from jax import lax
from jax.experimental import pallas as pl
from jax.experimental.pallas import tpu as pltpu
from jax.experimental.pallas import tpu_sc as plsc

B, N, VOCAB, EMB = 16, 1024, 4096, 192


_NT = 1024


def _argmax_body(x_ref, tok_ref):
    xb = x_ref[0]
    m = jnp.max(xb, axis=-1, keepdims=True)
    iota = lax.broadcasted_iota(jnp.int32, xb.shape, 1)
    idx = jnp.min(jnp.where(xb == m, iota, VOCAB), axis=-1)
    tok_ref[0, 0] = idx.astype(jnp.int32)


def _argmax_tokens(x):
    return pl.pallas_call(
        _argmax_body,
        grid=(B, N // _NT),
        in_specs=[pl.BlockSpec((1, _NT, VOCAB), lambda b, n: (b, n, 0))],
        out_specs=pl.BlockSpec((1, 1, _NT), lambda b, n: (b, 0, n)),
        out_shape=jax.ShapeDtypeStruct((B, 1, N), jnp.int32),
    )(x)


_NC, _NS, _L = 2, 16, 16
_NW = _NC * _NS
_ROWS = B * N
_BPW = _ROWS // _NW
_CHUNK = 128
_NCHUNK = _BPW // _CHUNK


def _make_sc_gather():
    mesh = plsc.VectorSubcoreMesh(core_axis_name="c", subcore_axis_name="s")

    @functools.partial(
        pl.kernel,
        mesh=mesh,
        out_type=jax.ShapeDtypeStruct((B, N, EMB), jnp.float32),
        scratch_types=[
            pltpu.VMEM((_NCHUNK, _CHUNK), jnp.int32),
            pltpu.VMEM((_BPW, EMB), jnp.float32),
            pltpu.SemaphoreType.DMA,
        ],
        compiler_params=pltpu.CompilerParams(use_tc_tiling_on_sc=False),
    )
    def sc_gather(table_hbm, idx_hbm, out_hbm, idx_v, rows_v, sem):
        wid = lax.axis_index("s") * _NC + lax.axis_index("c")
        b = wid // 2
        noff = (wid % 2) * _BPW
        for j in range(_NCHUNK):
            pltpu.sync_copy(
                idx_hbm.at[b, 0, pl.ds(noff + j * _CHUNK, _CHUNK)],
                idx_v.at[j],
            )
        copies = []
        for j in range(_NCHUNK):
            copies.append(
                pltpu.async_copy(
                    table_hbm.at[idx_v.at[j]],
                    rows_v.at[pl.ds(j * _CHUNK, _CHUNK)],
                    sem,
                )
            )
        for c in copies:
            c.wait()
        pltpu.sync_copy(rows_v, out_hbm.at[b, pl.ds(noff, _BPW)])

    return sc_gather


_SC_GATHER_CACHE = []


def kernel(x, dictionary):
    tokens = _argmax_tokens(x)
    if not _SC_GATHER_CACHE:
        _SC_GATHER_CACHE.append(_make_sc_gather())
    return _SC_GATHER_CACHE[0](dictionary, tokens)
